# baseline (speedup 1.0000x reference)
import jax
import jax.numpy as jnp
from jax import lax
from jax.experimental import pallas as pl
from jax.experimental.pallas import tpu as pltpu

_NENT = 100000
_BATCH = 1024
_KBLK = 1000
_NSTEPS = _NENT // _KBLK

_DN0 = (((0,), (0,)), ((), ()))


def _fused_kernel(hT_ref, tT_ref, wehT_ref, wetT_ref,
                  relsT_ref, wrT_ref, wriT_ref,
                  b_eh_ref, b_et_ref, b_r_ref, b_ri_ref,
                  out_ref, acc_h, acc_t):
    k = pl.program_id(0)

    w = jnp.concatenate([wehT_ref[...], wetT_ref[...]], axis=1)

    ph = lax.dot_general(hT_ref[...], w, _DN0,
                         preferred_element_type=jnp.float32)
    pt = lax.dot_general(tT_ref[...], w, _DN0,
                         preferred_element_type=jnp.float32)

    @pl.when(k == 0)
    def _():
        acc_h[...] = ph
        acc_t[...] = pt

    @pl.when(k > 0)
    def _():
        acc_h[...] += ph
        acc_t[...] += pt

    @pl.when(k == _NSTEPS - 1)
    def _():
        r = lax.dot_general(relsT_ref[...], wrT_ref[...], _DN0,
                            preferred_element_type=jnp.float32) + b_r_ref[...]
        ri = lax.dot_general(relsT_ref[...], wriT_ref[...], _DN0,
                             preferred_element_type=jnp.float32) + b_ri_ref[...]
        hh = acc_h[:, :64] + b_eh_ref[...]
        th = acc_h[:, 64:] + b_et_ref[...]
        ht = acc_t[:, :64] + b_eh_ref[...]
        tt = acc_t[:, 64:] + b_et_ref[...]
        s1 = jnp.sum(hh * r * tt, axis=1)
        s2 = jnp.sum(ht * ri * th, axis=1)
        out_ref[...] = jnp.clip((s1 + s2) * 0.5, -20.0, 20.0)[:, None]


def kernel(heads, rels, tails, W_eh, b_eh, W_et, b_et, W_r, b_r, W_ri, b_ri):
    out = pl.pallas_call(
        _fused_kernel,
        grid=(_NSTEPS,),
        in_specs=[
            pl.BlockSpec((_KBLK, _BATCH), lambda k: (k, 0)),
            pl.BlockSpec((_KBLK, _BATCH), lambda k: (k, 0)),
            pl.BlockSpec((_KBLK, 64), lambda k: (k, 0)),
            pl.BlockSpec((_KBLK, 64), lambda k: (k, 0)),
            pl.BlockSpec((1000, _BATCH), lambda k: (0, 0)),
            pl.BlockSpec((1000, 64), lambda k: (0, 0)),
            pl.BlockSpec((1000, 64), lambda k: (0, 0)),
            pl.BlockSpec((1, 64), lambda k: (0, 0)),
            pl.BlockSpec((1, 64), lambda k: (0, 0)),
            pl.BlockSpec((1, 64), lambda k: (0, 0)),
            pl.BlockSpec((1, 64), lambda k: (0, 0)),
        ],
        out_specs=pl.BlockSpec((_BATCH, 1), lambda k: (0, 0)),
        out_shape=jax.ShapeDtypeStruct((_BATCH, 1), jnp.float32),
        scratch_shapes=[pltpu.VMEM((_BATCH, 128), jnp.float32),
                        pltpu.VMEM((_BATCH, 128), jnp.float32)],
    )(heads.T, tails.T, W_eh.T, W_et.T, rels.T, W_r.T, W_ri.T,
      b_eh[None, :], b_et[None, :], b_r[None, :], b_ri[None, :])
    return out[:, 0]


# speedup vs baseline: 1.0563x; 1.0563x over previous
import jax
import jax.numpy as jnp
from jax import lax
from jax.experimental import pallas as pl
from jax.experimental.pallas import tpu as pltpu

_NENT = 100000
_BATCH = 1024
_KBLK = 2000
_NSTEPS = _NENT // _KBLK

_DN0 = (((0,), (0,)), ((), ()))


def _fused_kernel(hT_ref, tT_ref, wehT_ref, wetT_ref,
                  relsT_ref, wrT_ref, wriT_ref,
                  b_eh_ref, b_et_ref, b_r_ref, b_ri_ref,
                  out_ref, acc_h, acc_t):
    k = pl.program_id(0)

    w = jnp.concatenate([wehT_ref[...], wetT_ref[...]], axis=1)

    ph = lax.dot_general(hT_ref[...], w, _DN0,
                         preferred_element_type=jnp.float32)
    pt = lax.dot_general(tT_ref[...], w, _DN0,
                         preferred_element_type=jnp.float32)

    @pl.when(k == 0)
    def _():
        acc_h[...] = ph
        acc_t[...] = pt

    @pl.when(k > 0)
    def _():
        acc_h[...] += ph
        acc_t[...] += pt

    @pl.when(k == _NSTEPS - 1)
    def _():
        r = lax.dot_general(relsT_ref[...], wrT_ref[...], _DN0,
                            preferred_element_type=jnp.float32) + b_r_ref[...]
        ri = lax.dot_general(relsT_ref[...], wriT_ref[...], _DN0,
                             preferred_element_type=jnp.float32) + b_ri_ref[...]
        hh = acc_h[:, :64] + b_eh_ref[...]
        th = acc_h[:, 64:] + b_et_ref[...]
        ht = acc_t[:, :64] + b_eh_ref[...]
        tt = acc_t[:, 64:] + b_et_ref[...]
        s1 = jnp.sum(hh * r * tt, axis=1)
        s2 = jnp.sum(ht * ri * th, axis=1)
        out_ref[...] = jnp.clip((s1 + s2) * 0.5, -20.0, 20.0)[:, None]


def kernel(heads, rels, tails, W_eh, b_eh, W_et, b_et, W_r, b_r, W_ri, b_ri):
    out = pl.pallas_call(
        _fused_kernel,
        grid=(_NSTEPS,),
        in_specs=[
            pl.BlockSpec((_KBLK, _BATCH), lambda k: (k, 0)),
            pl.BlockSpec((_KBLK, _BATCH), lambda k: (k, 0)),
            pl.BlockSpec((_KBLK, 64), lambda k: (k, 0)),
            pl.BlockSpec((_KBLK, 64), lambda k: (k, 0)),
            pl.BlockSpec((1000, _BATCH), lambda k: (0, 0)),
            pl.BlockSpec((1000, 64), lambda k: (0, 0)),
            pl.BlockSpec((1000, 64), lambda k: (0, 0)),
            pl.BlockSpec((1, 64), lambda k: (0, 0)),
            pl.BlockSpec((1, 64), lambda k: (0, 0)),
            pl.BlockSpec((1, 64), lambda k: (0, 0)),
            pl.BlockSpec((1, 64), lambda k: (0, 0)),
        ],
        out_specs=pl.BlockSpec((_BATCH, 1), lambda k: (0, 0)),
        out_shape=jax.ShapeDtypeStruct((_BATCH, 1), jnp.float32),
        scratch_shapes=[pltpu.VMEM((_BATCH, 128), jnp.float32),
                        pltpu.VMEM((_BATCH, 128), jnp.float32)],
    )(heads.T, tails.T, W_eh.T, W_et.T, rels.T, W_r.T, W_ri.T,
      b_eh[None, :], b_et[None, :], b_r[None, :], b_ri[None, :])
    return out[:, 0]
